# Initial kernel scaffold; baseline (speedup 1.0000x reference)
import jax
import jax.numpy as jnp
from jax.experimental import pallas as pl


def kernel(edge_index, edge_vals, user_embeds, item_embeds, keep_rate):
    raise NotImplementedError("write your pallas kernel here")


# speedup vs baseline: 3.3109x; 3.3109x over previous
import dataclasses
import functools

import jax
import jax.numpy as jnp
from jax import lax
from jax.experimental import pallas as pl
from jax.experimental.pallas import tpu as pltpu
from jax.experimental.pallas import tpu_sc as plsc

_USER_NUM = 6000
_ITEM_NUM = 4000
_N = _USER_NUM + _ITEM_NUM
_D = 128
_E = 320000
_LAYERS = 3

_NC = 2
_NS = 16
_NWORK = _NC * _NS
_LANES = 16
_W = 128
_NWIN = -(-(_E // _NWORK) // _W)
_EPAD = _NWORK * _NWIN * _W
_NPAD = 10240
_ROWS_PER_TILE = _NPAD // _NS

_mesh = plsc.VectorSubcoreMesh(
    core_axis_name="c", subcore_axis_name="s", num_cores=_NC, num_subcores=_NS
)

_sc_params = pltpu.CompilerParams()
if "needs_layout_passes" in pltpu.CompilerParams.__dataclass_fields__:
    _sc_params = dataclasses.replace(_sc_params, needs_layout_passes=False)


def _sc_layer(emb, src_w, dst_w, val_w):
    @functools.partial(
        pl.kernel,
        out_type=jax.ShapeDtypeStruct((_NC, _NPAD, _D), jnp.float32),
        mesh=_mesh,
        compiler_params=_sc_params,
        scratch_types=[
            pltpu.VMEM((_NWIN, _W), jnp.int32),
            pltpu.VMEM((_NWIN, _W), jnp.int32),
            pltpu.VMEM((_NWIN, _W), jnp.float32),
            pltpu.VMEM((_W, _D), jnp.float32),
            pltpu.VMEM_SHARED((_NPAD, _D), jnp.float32),
        ],
    )
    def layer(emb_hbm, src_hbm, dst_hbm, val_hbm, out_hbm,
              src_v, dst_v, val_v, rows_v, acc_sh):
        c = lax.axis_index("c")
        s = lax.axis_index("s")
        w = c * _NS + s

        pltpu.sync_copy(src_hbm.at[w], src_v)
        pltpu.sync_copy(dst_hbm.at[w], dst_v)
        pltpu.sync_copy(val_hbm.at[w], val_v)

        @pl.loop(0, _W)
        def _zero_rows(r):
            for c8 in range(_D // _LANES):
                rows_v[r, pl.ds(c8 * _LANES, _LANES)] = jnp.zeros(
                    (_LANES,), jnp.float32)

        base = s * _ROWS_PER_TILE
        for k in range(_ROWS_PER_TILE // _W):
            pltpu.sync_copy(rows_v.at[pl.ds(0, _W)],
                            acc_sh.at[pl.ds(base + k * _W, _W)])
        plsc.subcore_barrier()

        @pl.loop(0, _NWIN)
        def _window(j):
            pltpu.sync_copy(emb_hbm.at[src_v.at[j]], rows_v)

            @pl.loop(0, _W)
            def _scale(r):
                vv = plsc.load_gather(
                    val_v,
                    [jnp.full((_LANES,), j, jnp.int32),
                     jnp.full((_LANES,), r, jnp.int32)],
                )
                for c8 in range(_D // _LANES):
                    sl = pl.ds(c8 * _LANES, _LANES)
                    rows_v[r, sl] = rows_v[r, sl] * vv

            pltpu.sync_copy(rows_v, acc_sh.at[dst_v.at[j]], add=True)

        plsc.subcore_barrier()

        for k in range(_ROWS_PER_TILE // _W):
            pltpu.sync_copy(acc_sh.at[pl.ds(base + k * _W, _W)],
                            out_hbm.at[c, pl.ds(base + k * _W, _W)])

    return layer(emb, src_w, dst_w, val_w)


def _combine(partials, total_prev):
    def body(p_ref, t_ref, emb_ref, tot_ref):
        e = p_ref[0] + p_ref[1]
        emb_ref[...] = e
        tot_ref[...] = t_ref[...] + e

    return pl.pallas_call(
        body,
        out_shape=(jax.ShapeDtypeStruct((_NPAD, _D), jnp.float32),
                   jax.ShapeDtypeStruct((_NPAD, _D), jnp.float32)),
    )(partials, total_prev)


def _finalize(partials, total_prev):
    def body(p_ref, t_ref, o_ref):
        o_ref[...] = (t_ref[...] + p_ref[0] + p_ref[1]) * 0.25

    return pl.pallas_call(
        body,
        out_shape=jax.ShapeDtypeStruct((_NPAD, _D), jnp.float32),
    )(partials, total_prev)


def kernel(edge_index, edge_vals, user_embeds, item_embeds, keep_rate):
    del keep_rate
    emb0 = jnp.concatenate(
        [user_embeds, item_embeds,
         jnp.zeros((_NPAD - _N, _D), jnp.float32)], axis=0)
    dst = edge_index[0]
    src = edge_index[1]
    pad = _EPAD - _E
    src_w = jnp.pad(src, (0, pad)).reshape(_NWORK, _NWIN, _W)
    dst_w = jnp.pad(dst, (0, pad)).reshape(_NWORK, _NWIN, _W)
    val_w = jnp.pad(edge_vals, (0, pad)).reshape(_NWORK, _NWIN, _W)

    total = emb0
    emb = emb0
    for layer in range(_LAYERS):
        p = _sc_layer(emb, src_w, dst_w, val_w)
        if layer < _LAYERS - 1:
            emb, total = _combine(p, total)
        else:
            total = _finalize(p, total)
    return total[:_USER_NUM], total[_USER_NUM:_N]
